# Initial kernel scaffold; baseline (speedup 1.0000x reference)
import jax
import jax.numpy as jnp
from jax.experimental import pallas as pl


def kernel(node_emb, edge_index, node_ids, W_hop0, b_hop0, W_hop1, b_hop1, W_out, b_out):
    raise NotImplementedError("write your pallas kernel here")


# speedup vs baseline: 50.2395x; 50.2395x over previous
import functools

import jax
import jax.numpy as jnp
from jax import lax
from jax.experimental import pallas as pl
from jax.experimental.pallas import tpu as pltpu
from jax.experimental.pallas import tpu_sc as plsc

N_NODES = 10000
E = 320000
D = 128
H = 128
B = 1024

NS = 16
NPAD = 10240
EP = E // NS
SL = NPAD // NS
L = 16

_mesh = plsc.VectorSubcoreMesh(
    core_axis_name="c", subcore_axis_name="s", num_cores=1, num_subcores=NS
)


@functools.partial(
    pl.kernel,
    out_type=jax.ShapeDtypeStruct((2 * NPAD,), jnp.float32),
    mesh=_mesh,
    compiler_params=pltpu.CompilerParams(needs_layout_passes=False),
    scratch_types=[
        pltpu.VMEM((EP,), jnp.int32),
        pltpu.VMEM((EP,), jnp.int32),
        pltpu.VMEM((NPAD,), jnp.int32),
        pltpu.VMEM((NPAD,), jnp.int32),
        pltpu.VMEM((NPAD,), jnp.int32),
        pltpu.VMEM((B,), jnp.int32),
        pltpu.VMEM((NS, SL), jnp.int32),
        pltpu.VMEM((SL,), jnp.int32),
        pltpu.VMEM((SL,), jnp.int32),
        pltpu.VMEM((SL,), jnp.float32),
        pltpu.VMEM((SL,), jnp.float32),
        pltpu.VMEM_SHARED((NS, NPAD), jnp.int32),
        pltpu.VMEM_SHARED((NPAD,), jnp.int32),
    ],
)
def _sc_bfs_masks(edges_hbm, ids_hbm, out_hbm,
                  src_v, dst_v, mask0_v, f1_v, f2_v, ids_v,
                  slice_v, comb1_v, comb2_v, o0_v, o1_v,
                  stage_sh, comb_sh):
    tid = lax.axis_index("s")
    ebase = pl.multiple_of(tid * EP, 8)
    nbase = pl.multiple_of(tid * SL, 8)

    pltpu.sync_copy(edges_hbm.at[pl.ds(ebase, EP)], src_v)
    pltpu.sync_copy(edges_hbm.at[pl.ds(E + ebase, EP)], dst_v)
    pltpu.sync_copy(ids_hbm, ids_v)

    zeros = jnp.zeros((L,), jnp.int32)
    ones = jnp.ones((L,), jnp.int32)

    def zero_body(i, _):
        off = pl.multiple_of(i * L, 8)
        mask0_v[pl.ds(off, L)] = zeros
        f1_v[pl.ds(off, L)] = zeros
        f2_v[pl.ds(off, L)] = zeros
        return 0

    lax.fori_loop(0, NPAD // L, zero_body, 0)

    def seed_body(i, _):
        off = pl.multiple_of(i * L, 8)
        plsc.store_scatter(mask0_v, [ids_v[pl.ds(off, L)]], ones)
        return 0

    lax.fori_loop(0, B // L, seed_body, 0)

    def edge_pass(cur_ref, front_ref):
        def body(i, _):
            off = pl.multiple_of(i * L, 8)
            sv = src_v[pl.ds(off, L)]
            dv = dst_v[pl.ds(off, L)]
            fr = plsc.load_gather(cur_ref, [sv])
            plsc.store_scatter(front_ref, [dv], ones, mask=fr > 0)
            return 0

        lax.fori_loop(0, EP // L, body, 0)

    def combine(front_ref, comb_slice_ref, full_ref):
        pltpu.sync_copy(front_ref, stage_sh.at[tid])
        plsc.subcore_barrier()
        pltpu.sync_copy(stage_sh.at[:, pl.ds(nbase, SL)], slice_v)

        def rbody(j, _):
            joff = pl.multiple_of(j * L, 8)
            acc = slice_v[0, pl.ds(joff, L)]
            for r in range(1, NS):
                acc = jnp.maximum(acc, slice_v[r, pl.ds(joff, L)])
            comb_slice_ref[pl.ds(joff, L)] = acc
            return 0

        lax.fori_loop(0, SL // L, rbody, 0)
        pltpu.sync_copy(comb_slice_ref, comb_sh.at[pl.ds(nbase, SL)])
        plsc.subcore_barrier()
        if full_ref is not None:
            pltpu.sync_copy(comb_sh, full_ref)
        plsc.subcore_barrier()

    edge_pass(mask0_v, f1_v)
    combine(f1_v, comb1_v, f1_v)
    edge_pass(f1_v, f2_v)
    combine(f2_v, comb2_v, None)

    def out_body(j, _):
        joff = pl.multiple_of(j * L, 8)
        a = comb1_v[pl.ds(joff, L)]
        u = jnp.maximum(a, comb2_v[pl.ds(joff, L)])
        o0_v[pl.ds(joff, L)] = a.astype(jnp.float32)
        o1_v[pl.ds(joff, L)] = u.astype(jnp.float32)
        return 0

    lax.fori_loop(0, SL // L, out_body, 0)
    pltpu.sync_copy(o0_v, out_hbm.at[pl.ds(nbase, SL)])
    pltpu.sync_copy(o1_v, out_hbm.at[pl.ds(pl.multiple_of(NPAD + tid * SL, 8), SL)])


def _tc_body(masks_ref, emb_ref, w0_ref, b0_ref, w1_ref, b1_ref,
             wo_ref, bo_ref, out_ref):
    hi = jax.lax.Precision.HIGHEST
    masks = masks_ref[...]
    cnt = jnp.sum(masks, axis=1)
    sums = lax.dot_general(masks[:, :N_NODES], emb_ref[...],
                           (((1,), (0,)), ((), ())), precision=hi)
    agg = jnp.where((cnt > 0)[:, None],
                    sums / jnp.maximum(cnt, 1.0)[:, None], 0.0)
    h0 = jnp.maximum(
        lax.dot_general(agg[0:1], w0_ref[...], (((1,), (1,)), ((), ())),
                        precision=hi) + b0_ref[...], 0.0)
    h1 = jnp.maximum(
        lax.dot_general(agg[1:2], w1_ref[...], (((1,), (1,)), ((), ())),
                        precision=hi) + b1_ref[...], 0.0)
    combined = jnp.concatenate([h0, h1], axis=1)
    row = lax.dot_general(combined, wo_ref[...], (((1,), (1,)), ((), ())),
                          precision=hi) + bo_ref[...]
    out_ref[...] = jnp.broadcast_to(row, (B, D))


_tc_head = pl.pallas_call(
    _tc_body,
    out_shape=jax.ShapeDtypeStruct((B, D), jnp.float32),
)


def kernel(node_emb, edge_index, node_ids, W_hop0, b_hop0, W_hop1, b_hop1,
           W_out, b_out):
    edges_flat = edge_index.reshape(-1)
    masks = _sc_bfs_masks(edges_flat, node_ids).reshape(2, NPAD)
    return _tc_head(masks, node_emb, W_hop0, b_hop0.reshape(1, H),
                    W_hop1, b_hop1.reshape(1, H), W_out, b_out.reshape(1, D))


# speedup vs baseline: 93.3541x; 1.8582x over previous
import functools

import jax
import jax.numpy as jnp
from jax import lax
from jax.experimental import pallas as pl
from jax.experimental.pallas import tpu as pltpu
from jax.experimental.pallas import tpu_sc as plsc

N_NODES = 10000
E = 320000
D = 128
H = 128
B = 1024

NS = 16
NPAD = 10240
EP = E // NS
SL = NPAD // NS
L = 16

_mesh = plsc.VectorSubcoreMesh(
    core_axis_name="c", subcore_axis_name="s", num_cores=1, num_subcores=NS
)


@functools.partial(
    pl.kernel,
    out_type=jax.ShapeDtypeStruct((2 * NPAD,), jnp.float32),
    mesh=_mesh,
    compiler_params=pltpu.CompilerParams(needs_layout_passes=False),
    scratch_types=[
        pltpu.VMEM((EP,), jnp.int32),
        pltpu.VMEM((EP,), jnp.int32),
        pltpu.VMEM((NPAD,), jnp.int32),
        pltpu.VMEM((NPAD,), jnp.int32),
        pltpu.VMEM((NPAD,), jnp.int32),
        pltpu.VMEM((B,), jnp.int32),
        pltpu.VMEM((NS, SL), jnp.int32),
        pltpu.VMEM((SL,), jnp.int32),
        pltpu.VMEM((SL,), jnp.int32),
        pltpu.VMEM((SL,), jnp.float32),
        pltpu.VMEM((SL,), jnp.float32),
        pltpu.VMEM_SHARED((NS, NPAD), jnp.int32),
        pltpu.VMEM_SHARED((NPAD,), jnp.int32),
        pltpu.SemaphoreType.DMA,
        pltpu.SemaphoreType.DMA,
        pltpu.SemaphoreType.DMA,
    ],
)
def _sc_bfs_masks(edges_hbm, ids_hbm, out_hbm,
                  src_v, dst_v, mask0_v, f1_v, f2_v, ids_v,
                  slice_v, comb1_v, comb2_v, o0_v, o1_v,
                  stage_sh, comb_sh, sem_s, sem_d, sem_i):
    tid = lax.axis_index("s")
    ebase = pl.multiple_of(tid * EP, 8)
    nbase = pl.multiple_of(tid * SL, 8)

    cp_s = pltpu.async_copy(edges_hbm.at[pl.ds(ebase, EP)], src_v, sem_s)
    cp_d = pltpu.async_copy(edges_hbm.at[pl.ds(E + ebase, EP)], dst_v, sem_d)
    cp_i = pltpu.async_copy(ids_hbm, ids_v, sem_i)

    zeros = jnp.zeros((L,), jnp.int32)
    ones = jnp.ones((L,), jnp.int32)

    @plsc.parallel_loop(0, NPAD, step=L, unroll=8)
    def _(i):
        off = pl.multiple_of(i, 8)
        mask0_v[pl.ds(off, L)] = zeros
        f1_v[pl.ds(off, L)] = zeros
        f2_v[pl.ds(off, L)] = zeros

    cp_i.wait()

    @plsc.parallel_loop(0, B, step=L, unroll=4)
    def _(i):
        off = pl.multiple_of(i, 8)
        plsc.store_scatter(mask0_v, [ids_v[pl.ds(off, L)]], ones)

    cp_s.wait()
    cp_d.wait()

    def edge_pass(cur_ref, front_ref):
        @plsc.parallel_loop(0, EP, step=L, unroll=8)
        def _(i):
            off = pl.multiple_of(i, 8)
            sv = src_v[pl.ds(off, L)]
            dv = dst_v[pl.ds(off, L)]
            fr = plsc.load_gather(cur_ref, [sv])
            plsc.store_scatter(front_ref, [dv], ones, mask=fr > 0)

    def combine(front_ref, comb_slice_ref):
        pltpu.sync_copy(front_ref, stage_sh.at[tid])
        plsc.subcore_barrier()
        pltpu.sync_copy(stage_sh.at[:, pl.ds(nbase, SL)], slice_v)

        @plsc.parallel_loop(0, SL, step=L, unroll=4)
        def _(j):
            joff = pl.multiple_of(j, 8)
            acc = slice_v[0, pl.ds(joff, L)]
            for r in range(1, NS):
                acc = jnp.maximum(acc, slice_v[r, pl.ds(joff, L)])
            comb_slice_ref[pl.ds(joff, L)] = acc

    edge_pass(mask0_v, f1_v)
    combine(f1_v, comb1_v)
    pltpu.sync_copy(comb1_v, comb_sh.at[pl.ds(nbase, SL)])
    plsc.subcore_barrier()
    pltpu.sync_copy(comb_sh, f1_v)

    edge_pass(f1_v, f2_v)
    combine(f2_v, comb2_v)

    @plsc.parallel_loop(0, SL, step=L, unroll=4)
    def _(j):
        joff = pl.multiple_of(j, 8)
        a = comb1_v[pl.ds(joff, L)]
        u = jnp.maximum(a, comb2_v[pl.ds(joff, L)])
        o0_v[pl.ds(joff, L)] = a.astype(jnp.float32)
        o1_v[pl.ds(joff, L)] = u.astype(jnp.float32)

    pltpu.sync_copy(o0_v, out_hbm.at[pl.ds(nbase, SL)])
    pltpu.sync_copy(o1_v, out_hbm.at[pl.ds(pl.multiple_of(NPAD + tid * SL, 8), SL)])


def _tc_body(masks_ref, emb_ref, w0_ref, b0_ref, w1_ref, b1_ref,
             wo_ref, bo_ref, out_ref):
    hi = jax.lax.Precision.HIGHEST
    masks = masks_ref[...]
    cnt = jnp.sum(masks, axis=1)
    sums = lax.dot_general(masks[:, :N_NODES], emb_ref[...],
                           (((1,), (0,)), ((), ())), precision=hi)
    agg = jnp.where((cnt > 0)[:, None],
                    sums / jnp.maximum(cnt, 1.0)[:, None], 0.0)
    h0 = jnp.maximum(
        lax.dot_general(agg[0:1], w0_ref[...], (((1,), (1,)), ((), ())),
                        precision=hi) + b0_ref[...], 0.0)
    h1 = jnp.maximum(
        lax.dot_general(agg[1:2], w1_ref[...], (((1,), (1,)), ((), ())),
                        precision=hi) + b1_ref[...], 0.0)
    combined = jnp.concatenate([h0, h1], axis=1)
    row = lax.dot_general(combined, wo_ref[...], (((1,), (1,)), ((), ())),
                          precision=hi) + bo_ref[...]
    out_ref[...] = jnp.broadcast_to(row, (B, D))


_tc_head = pl.pallas_call(
    _tc_body,
    out_shape=jax.ShapeDtypeStruct((B, D), jnp.float32),
)


def kernel(node_emb, edge_index, node_ids, W_hop0, b_hop0, W_hop1, b_hop1,
           W_out, b_out):
    masks = _sc_bfs_masks(edge_index.reshape(-1), node_ids).reshape(2, NPAD)
    return _tc_head(masks, node_emb, W_hop0, b_hop0.reshape(1, H),
                    W_hop1, b_hop1.reshape(1, H), W_out, b_out.reshape(1, D))
